# baseline (speedup 1.0000x reference)
import functools

import jax
import jax.numpy as jnp
from jax import lax
from jax.experimental import pallas as pl
from jax.experimental.pallas import tpu as pltpu
from jax.experimental.pallas import tpu_sc as plsc

_N = 52
_E = 832
_WIN = 5
_OUT = 128
_LANES = 16
_EG = _E // _LANES


def _sc_edge_scatter_body(zeros_hbm, src_hbm, dst_hbm, ew_hbm, out_hbm,
                          acc_v, src_v, dst_v, ew_v):
    cid = lax.axis_index("c")
    sid = lax.axis_index("s")

    @pl.when(jnp.logical_and(cid == 0, sid == 0))
    def _():
        pltpu.sync_copy(zeros_hbm, acc_v)
        pltpu.sync_copy(src_hbm, src_v)
        pltpu.sync_copy(dst_hbm, dst_v)
        pltpu.sync_copy(ew_hbm, ew_v)

        def body(g, carry):
            off = pl.multiple_of(g * _LANES, _LANES)
            s = src_v[pl.ds(off, _LANES)]
            d = dst_v[pl.ds(off, _LANES)]
            w = ew_v[pl.ds(off, _LANES)]
            plsc.addupdate_scatter(acc_v, [d * _N + s], w)
            return carry

        lax.fori_loop(0, _EG, body, 0)
        pltpu.sync_copy(acc_v, out_hbm)


@functools.cache
def _sc_edge_scatter():
    return pl.kernel(
        _sc_edge_scatter_body,
        out_type=jax.ShapeDtypeStruct((_N * _N,), jnp.float32),
        mesh=plsc.VectorSubcoreMesh(core_axis_name="c", subcore_axis_name="s"),
        compiler_params=pltpu.CompilerParams(needs_layout_passes=False),
        scratch_types=[
            pltpu.VMEM((_N * _N,), jnp.float32),
            pltpu.VMEM((_E,), jnp.int32),
            pltpu.VMEM((_E,), jnp.int32),
            pltpu.VMEM((_E,), jnp.float32),
        ],
    )


def _tc_temporal_body(x_ref, w_ref, b_ref, aw_ref, h_ref):
    hs = [jnp.dot(x_ref[t], w_ref[t], preferred_element_type=jnp.float32)
          for t in range(_WIN)]
    att = aw_ref[...]
    ss = [jnp.sum(h * att, axis=1, keepdims=True) for h in hs]
    m = ss[0]
    for s in ss[1:]:
        m = jnp.maximum(m, s)
    es = [jnp.exp(s - m) for s in ss]
    z = es[0]
    for e in es[1:]:
        z = z + e
    h = es[0] * hs[0]
    for t in range(1, _WIN):
        h = h + es[t] * hs[t]
    h_ref[...] = h / z + b_ref[...]


def _tc_gcn_body(h_ref, W1_ref, b1_ref, W2_ref, b2_ref,
                 g1_ref, be1_ref, g2_ref, be2_ref, lw_ref, lb_ref, acc_ref,
                 o_ref):
    a_raw = acc_ref[...]
    deg = jnp.sum(a_raw, axis=1, keepdims=True) + 1.0
    dinv = lax.rsqrt(deg)
    rr = lax.broadcasted_iota(jnp.int32, (_N, _N), 0)
    cc = lax.broadcasted_iota(jnp.int32, (_N, _N), 1)
    a_n = jnp.where(rr == cc, a_raw + 1.0, a_raw)

    def gcn(hin, W_r, bb_r):
        hw = jnp.dot(hin, W_r[...], preferred_element_type=jnp.float32)
        agg = jnp.dot(a_n, dinv * hw, preferred_element_type=jnp.float32)
        return dinv * agg + bb_r[...]

    def norm_relu(v, g_r, be_r):
        mean = jnp.mean(v, axis=1, keepdims=True)
        cen = v - mean
        var = jnp.mean(cen * cen, axis=1, keepdims=True)
        vn = cen * lax.rsqrt(var + 1e-5) * g_r[...] + be_r[...]
        return jnp.maximum(vn, 0.0)

    h1 = norm_relu(gcn(h_ref[...], W1_ref, b1_ref), g1_ref, be1_ref)
    h2 = norm_relu(gcn(h1, W2_ref, b2_ref), g2_ref, be2_ref)
    o_ref[...] = (jnp.dot(h2, lw_ref[...], preferred_element_type=jnp.float32)
                  + lb_ref[...])


def _tc_temporal_call(args, interpret=False):
    return pl.pallas_call(
        _tc_temporal_body,
        out_shape=jax.ShapeDtypeStruct((_N, 512), jnp.float32),
        interpret=interpret,
    )(*args)


def _tc_gcn_call(args, interpret=False):
    return pl.pallas_call(
        _tc_gcn_body,
        out_shape=jax.ShapeDtypeStruct((_N, _OUT), jnp.float32),
        interpret=interpret,
    )(*args)


def kernel(x, edge_index, edge_weight, weight, bias, attn_w, W1, b1, W2, b2,
           bn1_g, bn1_b, bn2_g, bn2_b, lin_W, lin_b):
    ei = jnp.asarray(edge_index, jnp.int32)
    zeros = jnp.zeros((_N * _N,), jnp.float32)
    acc = _sc_edge_scatter()(zeros, ei[0], ei[1],
                             jnp.asarray(edge_weight, jnp.float32))
    acc = acc.reshape(_N, _N)
    h = _tc_temporal_call((x, weight, bias.reshape(1, -1),
                           attn_w.reshape(1, -1)))
    return _tc_gcn_call((
        h,
        W1, b1.reshape(1, -1), W2, b2.reshape(1, -1),
        bn1_g.reshape(-1, 1), bn1_b.reshape(-1, 1),
        bn2_g.reshape(-1, 1), bn2_b.reshape(-1, 1),
        lin_W, lin_b.reshape(1, -1), acc,
    ))


# speedup vs baseline: 1.1857x; 1.1857x over previous
import functools

import jax
import jax.numpy as jnp
from jax import lax
from jax.experimental import pallas as pl
from jax.experimental.pallas import tpu as pltpu
from jax.experimental.pallas import tpu_sc as plsc

_N = 52
_NP = 64
_E = 832
_WIN = 5
_HID = 512
_OUT = 128
_LANES = 16
_EG = _E // _LANES


def _sc_edge_scatter_body(edge_hbm, ew_hbm, out_hbm, acc_v, edge_v, ew_v):
    cid = lax.axis_index("c")
    sid = lax.axis_index("s")

    @pl.when(jnp.logical_and(cid == 0, sid == 0))
    def _():
        pltpu.sync_copy(edge_hbm, edge_v)
        pltpu.sync_copy(ew_hbm, ew_v)
        zero = jnp.zeros((_LANES,), jnp.float32)
        for r in range(_N):
            for c in range(_NP // _LANES):
                acc_v[r, pl.ds(c * _LANES, _LANES)] = zero
        for g in range(_EG):
            s = edge_v[0, pl.ds(g * _LANES, _LANES)]
            d = edge_v[1, pl.ds(g * _LANES, _LANES)]
            w = ew_v[pl.ds(g * _LANES, _LANES)]
            plsc.addupdate_scatter(acc_v, [d, s], w)
        pltpu.sync_copy(acc_v, out_hbm)


@functools.cache
def _sc_edge_scatter():
    return pl.kernel(
        _sc_edge_scatter_body,
        out_type=jax.ShapeDtypeStruct((_N, _NP), jnp.float32),
        mesh=plsc.VectorSubcoreMesh(core_axis_name="c", subcore_axis_name="s"),
        compiler_params=pltpu.CompilerParams(needs_layout_passes=False),
        scratch_types=[
            pltpu.VMEM((_N, _NP), jnp.float32),
            pltpu.VMEM((2, _E), jnp.int32),
            pltpu.VMEM((_E,), jnp.float32),
        ],
    )


def _tc_temporal_body(x_ref, w_ref, b_ref, aw_ref, h_ref):
    hs = [jnp.dot(x_ref[t], w_ref[t], preferred_element_type=jnp.float32)
          for t in range(_WIN)]
    att = aw_ref[...]
    ss = [jnp.sum(h * att, axis=1, keepdims=True) for h in hs]
    m = ss[0]
    for s in ss[1:]:
        m = jnp.maximum(m, s)
    es = [jnp.exp(s - m) for s in ss]
    z = es[0]
    for e in es[1:]:
        z = z + e
    h = es[0] * hs[0]
    for t in range(1, _WIN):
        h = h + es[t] * hs[t]
    h_ref[...] = h / z + b_ref[...]


def _tc_gcn_body(h_ref, W1_ref, b1_ref, W2_ref, b2_ref, bn_ref,
                 lw_ref, lb_ref, acc_ref, o_ref):
    a_raw = acc_ref[:, :_N]
    deg = jnp.sum(a_raw, axis=1, keepdims=True) + 1.0
    dinv = lax.rsqrt(deg)
    rr = lax.broadcasted_iota(jnp.int32, (_N, _N), 0)
    cc = lax.broadcasted_iota(jnp.int32, (_N, _N), 1)
    a_n = jnp.where(rr == cc, a_raw + 1.0, a_raw)

    def gcn(hin, W_r, bb_r):
        hw = jnp.dot(hin, W_r[...], preferred_element_type=jnp.float32)
        agg = jnp.dot(a_n, dinv * hw, preferred_element_type=jnp.float32)
        return dinv * agg + bb_r[...]

    def norm_relu(v, g, be):
        mean = jnp.mean(v, axis=1, keepdims=True)
        cen = v - mean
        var = jnp.mean(cen * cen, axis=1, keepdims=True)
        vn = cen * lax.rsqrt(var + 1e-5) * g + be
        return jnp.maximum(vn, 0.0)

    bn = bn_ref[...]
    h1 = norm_relu(gcn(h_ref[...], W1_ref, b1_ref),
                   bn[:, 0:1], bn[:, 1:2])
    h2 = norm_relu(gcn(h1, W2_ref, b2_ref), bn[:, 2:3], bn[:, 3:4])
    o_ref[...] = (jnp.dot(h2, lw_ref[...], preferred_element_type=jnp.float32)
                  + lb_ref[...])


def _tc_temporal_call(args, interpret=False):
    return pl.pallas_call(
        _tc_temporal_body,
        out_shape=jax.ShapeDtypeStruct((_N, _HID), jnp.float32),
        interpret=interpret,
    )(*args)


def _tc_gcn_call(args, interpret=False):
    return pl.pallas_call(
        _tc_gcn_body,
        out_shape=jax.ShapeDtypeStruct((_N, _OUT), jnp.float32),
        interpret=interpret,
    )(*args)


def kernel(x, edge_index, edge_weight, weight, bias, attn_w, W1, b1, W2, b2,
           bn1_g, bn1_b, bn2_g, bn2_b, lin_W, lin_b):
    acc = _sc_edge_scatter()(jnp.asarray(edge_index, jnp.int32),
                             jnp.asarray(edge_weight, jnp.float32))
    h = _tc_temporal_call((x, weight, bias.reshape(1, -1),
                           attn_w.reshape(1, -1)))
    bn = jnp.stack([bn1_g, bn1_b, bn2_g, bn2_b], axis=1)
    return _tc_gcn_call((
        h,
        W1, b1.reshape(1, -1), W2, b2.reshape(1, -1), bn,
        lin_W, lin_b.reshape(1, -1), acc,
    ))


# speedup vs baseline: 1.2245x; 1.0328x over previous
import functools

import jax
import jax.numpy as jnp
from jax import lax
from jax.experimental import pallas as pl
from jax.experimental.pallas import tpu as pltpu
from jax.experimental.pallas import tpu_sc as plsc

_N = 52
_NP = 64
_E = 832
_WIN = 5
_HID = 512
_OUT = 128
_LANES = 16
_EG = _E // _LANES


def _sc_edge_scatter_body(edge_hbm, ew_hbm, out_hbm, acc_v, edge_v, ew_v,
                          sem1, sem2):
    cid = lax.axis_index("c")
    sid = lax.axis_index("s")

    @pl.when(jnp.logical_and(cid == 0, sid == 0))
    def _():
        cp_edge = pltpu.make_async_copy(edge_hbm, edge_v, sem1)
        cp_ew = pltpu.make_async_copy(ew_hbm, ew_v, sem2)
        cp_edge.start()
        cp_ew.start()
        zero = jnp.zeros((_LANES,), jnp.float32)
        for r in range(_N):
            for c in range(_NP // _LANES):
                acc_v[r, pl.ds(c * _LANES, _LANES)] = zero
        cp_edge.wait()
        cp_ew.wait()
        for g in range(_EG):
            s = edge_v[0, pl.ds(g * _LANES, _LANES)]
            d = edge_v[1, pl.ds(g * _LANES, _LANES)]
            w = ew_v[pl.ds(g * _LANES, _LANES)]
            plsc.addupdate_scatter(acc_v, [d, s], w)
        pltpu.sync_copy(acc_v, out_hbm)


@functools.cache
def _sc_edge_scatter():
    return pl.kernel(
        _sc_edge_scatter_body,
        out_type=jax.ShapeDtypeStruct((_N, _NP), jnp.float32),
        mesh=plsc.VectorSubcoreMesh(core_axis_name="c", subcore_axis_name="s"),
        compiler_params=pltpu.CompilerParams(needs_layout_passes=False),
        scratch_types=[
            pltpu.VMEM((_N, _NP), jnp.float32),
            pltpu.VMEM((2, _E), jnp.int32),
            pltpu.VMEM((_E,), jnp.float32),
            pltpu.SemaphoreType.DMA,
            pltpu.SemaphoreType.DMA,
        ],
    )


def _tc_temporal_body(x_ref, w_ref, b_ref, aw_ref, h_ref, hs_s):
    t = pl.program_id(0)
    hs_s[t] = jnp.dot(x_ref[0], w_ref[0], preferred_element_type=jnp.float32)

    @pl.when(t == _WIN - 1)
    def _():
        hs = [hs_s[i] for i in range(_WIN)]
        att = aw_ref[...]
        ss = [jnp.sum(h * att, axis=1, keepdims=True) for h in hs]
        m = ss[0]
        for s in ss[1:]:
            m = jnp.maximum(m, s)
        es = [jnp.exp(s - m) for s in ss]
        z = es[0]
        for e in es[1:]:
            z = z + e
        h = es[0] * hs[0]
        for i in range(1, _WIN):
            h = h + es[i] * hs[i]
        h_ref[...] = h / z + b_ref[...]


def _tc_gcn_body(h_ref, W1_ref, b1_ref, W2_ref, b2_ref,
                 g1_ref, be1_ref, g2_ref, be2_ref, lw_ref, lb_ref, acc_ref,
                 o_ref):
    a_raw = acc_ref[:, :_N]
    deg = jnp.sum(a_raw, axis=1, keepdims=True) + 1.0
    dinv = lax.rsqrt(deg)
    rr = lax.broadcasted_iota(jnp.int32, (_N, _N), 0)
    cc = lax.broadcasted_iota(jnp.int32, (_N, _N), 1)
    a_n = jnp.where(rr == cc, a_raw + 1.0, a_raw)

    bn_rows = jnp.concatenate(
        [g1_ref[...], be1_ref[...], g2_ref[...], be2_ref[...]], axis=0)
    bn = jnp.transpose(bn_rows)

    def gcn(hin, W_r, bb_r):
        hw = jnp.dot(hin, W_r[...], preferred_element_type=jnp.float32)
        agg = jnp.dot(a_n, dinv * hw, preferred_element_type=jnp.float32)
        return dinv * agg + bb_r[...]

    def norm_relu(v, g, be):
        mean = jnp.mean(v, axis=1, keepdims=True)
        cen = v - mean
        var = jnp.mean(cen * cen, axis=1, keepdims=True)
        vn = cen * lax.rsqrt(var + 1e-5) * g + be
        return jnp.maximum(vn, 0.0)

    h1 = norm_relu(gcn(h_ref[...], W1_ref, b1_ref), bn[:, 0:1], bn[:, 1:2])
    h2 = norm_relu(gcn(h1, W2_ref, b2_ref), bn[:, 2:3], bn[:, 3:4])
    o_ref[...] = (jnp.dot(h2, lw_ref[...], preferred_element_type=jnp.float32)
                  + lb_ref[...])


def _tc_temporal_call(args, interpret=False):
    return pl.pallas_call(
        _tc_temporal_body,
        grid=(_WIN,),
        in_specs=[
            pl.BlockSpec((1, _N, 512), lambda t: (t, 0, 0)),
            pl.BlockSpec((1, 512, _HID), lambda t: (t, 0, 0)),
            pl.BlockSpec((1, _HID), lambda t: (0, 0)),
            pl.BlockSpec((1, _HID), lambda t: (0, 0)),
        ],
        out_specs=pl.BlockSpec((_N, _HID), lambda t: (0, 0)),
        out_shape=jax.ShapeDtypeStruct((_N, _HID), jnp.float32),
        scratch_shapes=[pltpu.VMEM((_WIN, _N, _HID), jnp.float32)],
        interpret=interpret,
    )(*args)


def _tc_gcn_call(args, interpret=False):
    return pl.pallas_call(
        _tc_gcn_body,
        out_shape=jax.ShapeDtypeStruct((_N, _OUT), jnp.float32),
        interpret=interpret,
    )(*args)


def kernel(x, edge_index, edge_weight, weight, bias, attn_w, W1, b1, W2, b2,
           bn1_g, bn1_b, bn2_g, bn2_b, lin_W, lin_b):
    acc = _sc_edge_scatter()(jnp.asarray(edge_index, jnp.int32),
                             jnp.asarray(edge_weight, jnp.float32))
    h = _tc_temporal_call((x, weight, bias.reshape(1, -1),
                           attn_w.reshape(1, -1)))
    return _tc_gcn_call((
        h,
        W1, b1.reshape(1, -1), W2, b2.reshape(1, -1),
        bn1_g.reshape(1, -1), bn1_b.reshape(1, -1),
        bn2_g.reshape(1, -1), bn2_b.reshape(1, -1),
        lin_W, lin_b.reshape(1, -1), acc,
    ))


# speedup vs baseline: 1.2580x; 1.0273x over previous
import functools

import jax
import jax.numpy as jnp
from jax import lax
from jax.experimental import pallas as pl
from jax.experimental.pallas import tpu as pltpu
from jax.experimental.pallas import tpu_sc as plsc

_N = 52
_NP = 64
_E = 832
_WIN = 5
_HID = 512
_OUT = 128
_LANES = 16
_EG = _E // _LANES


def _sc_edge_scatter_body(edge_hbm, ew_hbm, out_hbm, acc_v, edge_v, ew_v,
                          sem1, sem2):
    cid = lax.axis_index("c")
    sid = lax.axis_index("s")

    @pl.when(jnp.logical_and(cid == 0, sid == 0))
    def _():
        cp_edge = pltpu.make_async_copy(edge_hbm, edge_v, sem1)
        cp_ew = pltpu.make_async_copy(ew_hbm, ew_v, sem2)
        cp_edge.start()
        cp_ew.start()
        zero = jnp.zeros((_LANES,), jnp.float32)

        def zbody(r, carry):
            for c in range(_NP // _LANES):
                acc_v[r, pl.ds(pl.multiple_of(c * _LANES, _LANES), _LANES)] \
                    = zero
            return carry

        lax.fori_loop(0, _N, zbody, 0)
        cp_edge.wait()
        cp_ew.wait()

        def sbody(g, carry):
            off = pl.multiple_of(g * _LANES, _LANES)
            s = edge_v[0, pl.ds(off, _LANES)]
            d = edge_v[1, pl.ds(off, _LANES)]
            w = ew_v[pl.ds(off, _LANES)]
            plsc.addupdate_scatter(acc_v, [d, s], w)
            return carry

        lax.fori_loop(0, _EG, sbody, 0)
        pltpu.sync_copy(acc_v, out_hbm)


@functools.cache
def _sc_edge_scatter():
    return pl.kernel(
        _sc_edge_scatter_body,
        out_type=jax.ShapeDtypeStruct((_N, _NP), jnp.float32),
        mesh=plsc.VectorSubcoreMesh(core_axis_name="c", subcore_axis_name="s"),
        compiler_params=pltpu.CompilerParams(needs_layout_passes=False),
        scratch_types=[
            pltpu.VMEM((_N, _NP), jnp.float32),
            pltpu.VMEM((2, _E), jnp.int32),
            pltpu.VMEM((_E,), jnp.float32),
            pltpu.SemaphoreType.DMA,
            pltpu.SemaphoreType.DMA,
        ],
    )


def _tc_temporal_body(x_ref, w_ref, b_ref, aw_ref, h_ref):
    hs = [jnp.dot(x_ref[t], w_ref[t], preferred_element_type=jnp.float32)
          for t in range(_WIN)]
    att = aw_ref[...]
    ss = [jnp.sum(h * att, axis=1, keepdims=True) for h in hs]
    m = ss[0]
    for s in ss[1:]:
        m = jnp.maximum(m, s)
    es = [jnp.exp(s - m) for s in ss]
    z = es[0]
    for e in es[1:]:
        z = z + e
    h = es[0] * hs[0]
    for t in range(1, _WIN):
        h = h + es[t] * hs[t]
    h_ref[...] = h / z + b_ref[...]


def _tc_gcn_body(h_ref, W1_ref, b1_ref, W2_ref, b2_ref,
                 g1_ref, be1_ref, g2_ref, be2_ref, lw_ref, lb_ref, acc_ref,
                 o_ref):
    a_raw = acc_ref[:, :_N]
    deg = jnp.sum(a_raw, axis=1, keepdims=True) + 1.0
    dinv = lax.rsqrt(deg)
    rr = lax.broadcasted_iota(jnp.int32, (_N, _N), 0)
    cc = lax.broadcasted_iota(jnp.int32, (_N, _N), 1)
    a_n = jnp.where(rr == cc, a_raw + 1.0, a_raw)

    bn_rows = jnp.concatenate(
        [g1_ref[...], be1_ref[...], g2_ref[...], be2_ref[...]], axis=0)
    bn = jnp.transpose(bn_rows)

    def gcn(hin, W_r, bb_r):
        hw = jnp.dot(hin, W_r[...], preferred_element_type=jnp.float32)
        agg = jnp.dot(a_n, dinv * hw, preferred_element_type=jnp.float32)
        return dinv * agg + bb_r[...]

    def norm_relu(v, g, be):
        mean = jnp.mean(v, axis=1, keepdims=True)
        cen = v - mean
        var = jnp.mean(cen * cen, axis=1, keepdims=True)
        vn = cen * lax.rsqrt(var + 1e-5) * g + be
        return jnp.maximum(vn, 0.0)

    h1 = norm_relu(gcn(h_ref[...], W1_ref, b1_ref), bn[:, 0:1], bn[:, 1:2])
    h2 = norm_relu(gcn(h1, W2_ref, b2_ref), bn[:, 2:3], bn[:, 3:4])
    o_ref[...] = (jnp.dot(h2, lw_ref[...], preferred_element_type=jnp.float32)
                  + lb_ref[...])


def _tc_temporal_call(args, interpret=False):
    return pl.pallas_call(
        _tc_temporal_body,
        out_shape=jax.ShapeDtypeStruct((_N, _HID), jnp.float32),
        interpret=interpret,
    )(*args)


def _tc_gcn_call(args, interpret=False):
    return pl.pallas_call(
        _tc_gcn_body,
        out_shape=jax.ShapeDtypeStruct((_N, _OUT), jnp.float32),
        interpret=interpret,
    )(*args)


def kernel(x, edge_index, edge_weight, weight, bias, attn_w, W1, b1, W2, b2,
           bn1_g, bn1_b, bn2_g, bn2_b, lin_W, lin_b):
    acc = _sc_edge_scatter()(jnp.asarray(edge_index, jnp.int32),
                             jnp.asarray(edge_weight, jnp.float32))
    h = _tc_temporal_call((x, weight, bias.reshape(1, -1),
                           attn_w.reshape(1, -1)))
    return _tc_gcn_call((
        h,
        W1, b1.reshape(1, -1), W2, b2.reshape(1, -1),
        bn1_g.reshape(1, -1), bn1_b.reshape(1, -1),
        bn2_g.reshape(1, -1), bn2_b.reshape(1, -1),
        lin_W, lin_b.reshape(1, -1), acc,
    ))


# speedup vs baseline: 1.2710x; 1.0103x over previous
import functools

import jax
import jax.numpy as jnp
from jax import lax
from jax.experimental import pallas as pl
from jax.experimental.pallas import tpu as pltpu
from jax.experimental.pallas import tpu_sc as plsc

_N = 52
_NP = 64
_E = 832
_WIN = 5
_HID = 512
_OUT = 128
_LANES = 16
_EG = _E // _LANES


def _sc_edge_scatter_body(edge_hbm, ew_hbm, out_hbm, acc_v, edge_v, ew_v,
                          sem1, sem2):
    cid = lax.axis_index("c")
    sid = lax.axis_index("s")

    @pl.when(sid == 0)
    def _():
        cp_edge = pltpu.make_async_copy(edge_hbm, edge_v, sem1)
        cp_ew = pltpu.make_async_copy(ew_hbm, ew_v, sem2)
        cp_edge.start()
        cp_ew.start()
        zero = jnp.zeros((_LANES,), jnp.float32)

        def zbody(r, carry):
            for c in range(_NP // _LANES):
                acc_v[r, pl.ds(pl.multiple_of(c * _LANES, _LANES), _LANES)] \
                    = zero
            return carry

        lax.fori_loop(0, _N, zbody, 0)
        cp_edge.wait()
        cp_ew.wait()
        base = cid * (_EG // 2)

        def sbody(g, carry):
            off = pl.multiple_of((base + g) * _LANES, _LANES)
            s = edge_v[0, pl.ds(off, _LANES)]
            d = edge_v[1, pl.ds(off, _LANES)]
            w = ew_v[pl.ds(off, _LANES)]
            plsc.addupdate_scatter(acc_v, [d, s], w)
            return carry

        lax.fori_loop(0, _EG // 2, sbody, 0)
        pltpu.sync_copy(acc_v, out_hbm.at[cid])


@functools.cache
def _sc_edge_scatter():
    return pl.kernel(
        _sc_edge_scatter_body,
        out_type=jax.ShapeDtypeStruct((2, _N, _NP), jnp.float32),
        mesh=plsc.VectorSubcoreMesh(core_axis_name="c", subcore_axis_name="s"),
        compiler_params=pltpu.CompilerParams(needs_layout_passes=False),
        scratch_types=[
            pltpu.VMEM((_N, _NP), jnp.float32),
            pltpu.VMEM((2, _E), jnp.int32),
            pltpu.VMEM((_E,), jnp.float32),
            pltpu.SemaphoreType.DMA,
            pltpu.SemaphoreType.DMA,
        ],
    )


def _tc_temporal_body(x_ref, w_ref, b_ref, aw_ref, h_ref):
    hs = [jnp.dot(x_ref[t], w_ref[t], preferred_element_type=jnp.float32)
          for t in range(_WIN)]
    att = aw_ref[...]
    ss = [jnp.sum(h * att, axis=1, keepdims=True) for h in hs]
    m = ss[0]
    for s in ss[1:]:
        m = jnp.maximum(m, s)
    es = [jnp.exp(s - m) for s in ss]
    z = es[0]
    for e in es[1:]:
        z = z + e
    h = es[0] * hs[0]
    for t in range(1, _WIN):
        h = h + es[t] * hs[t]
    h_ref[...] = h / z + b_ref[...]


def _tc_gcn_body(h_ref, W1_ref, b1_ref, W2_ref, b2_ref,
                 g1_ref, be1_ref, g2_ref, be2_ref, lw_ref, lb_ref, acc_ref,
                 o_ref):
    a_raw = acc_ref[0, :, :_N] + acc_ref[1, :, :_N]
    deg = jnp.sum(a_raw, axis=1, keepdims=True) + 1.0
    dinv = lax.rsqrt(deg)
    rr = lax.broadcasted_iota(jnp.int32, (_N, _N), 0)
    cc = lax.broadcasted_iota(jnp.int32, (_N, _N), 1)
    a_n = jnp.where(rr == cc, a_raw + 1.0, a_raw)

    bn_rows = jnp.concatenate(
        [g1_ref[...], be1_ref[...], g2_ref[...], be2_ref[...]], axis=0)
    bn = jnp.transpose(bn_rows)

    def gcn(hin, W_r, bb_r):
        hw = jnp.dot(hin, W_r[...], preferred_element_type=jnp.float32)
        agg = jnp.dot(a_n, dinv * hw, preferred_element_type=jnp.float32)
        return dinv * agg + bb_r[...]

    def norm_relu(v, g, be):
        mean = jnp.mean(v, axis=1, keepdims=True)
        cen = v - mean
        var = jnp.mean(cen * cen, axis=1, keepdims=True)
        vn = cen * lax.rsqrt(var + 1e-5) * g + be
        return jnp.maximum(vn, 0.0)

    h1 = norm_relu(gcn(h_ref[...], W1_ref, b1_ref), bn[:, 0:1], bn[:, 1:2])
    h2 = norm_relu(gcn(h1, W2_ref, b2_ref), bn[:, 2:3], bn[:, 3:4])
    o_ref[...] = (jnp.dot(h2, lw_ref[...], preferred_element_type=jnp.float32)
                  + lb_ref[...])


def _tc_temporal_call(args, interpret=False):
    return pl.pallas_call(
        _tc_temporal_body,
        out_shape=jax.ShapeDtypeStruct((_N, _HID), jnp.float32),
        interpret=interpret,
    )(*args)


def _tc_gcn_call(args, interpret=False):
    return pl.pallas_call(
        _tc_gcn_body,
        out_shape=jax.ShapeDtypeStruct((_N, _OUT), jnp.float32),
        interpret=interpret,
    )(*args)


def kernel(x, edge_index, edge_weight, weight, bias, attn_w, W1, b1, W2, b2,
           bn1_g, bn1_b, bn2_g, bn2_b, lin_W, lin_b):
    acc = _sc_edge_scatter()(jnp.asarray(edge_index, jnp.int32),
                             jnp.asarray(edge_weight, jnp.float32))
    h = _tc_temporal_call((x, weight, bias.reshape(1, -1),
                           attn_w.reshape(1, -1)))
    return _tc_gcn_call((
        h,
        W1, b1.reshape(1, -1), W2, b2.reshape(1, -1),
        bn1_g.reshape(1, -1), bn1_b.reshape(1, -1),
        bn2_g.reshape(1, -1), bn2_b.reshape(1, -1),
        lin_W, lin_b.reshape(1, -1), acc,
    ))
